# Initial kernel scaffold; baseline (speedup 1.0000x reference)
import functools

import jax
import jax.numpy as jnp
from jax import lax
from jax.experimental import pallas as pl
from jax.experimental.pallas import tpu as pltpu

N_PTS = 10000
N_PAD = 10240
M_SMP = 2500
M_PAD = 2560
K_NBR = 64
RADIUS = 0.2
D_FEAT = 128
D_HID = 128
_BIG_I32 = jnp.int32(2 ** 30)


def _extract_lane(ref, k):
    row = k // 128
    lane = k - row * 128
    rv = ref[pl.ds(row, 1), :]
    li = lax.broadcasted_iota(jnp.int32, (1, 128), 1)
    return jnp.sum(jnp.where(li == lane, rv, 0.0))


def _fps_body(posx_ref, posy_ref, posz_ref, poss_ref, idx_ref, dist_ref):
    ii = (lax.broadcasted_iota(jnp.int32, (80, 128), 0) * 128
          + lax.broadcasted_iota(jnp.int32, (80, 128), 1))
    dist_ref[...] = jnp.where(ii < N_PTS, jnp.inf, -jnp.inf).astype(jnp.float32)

    l4 = lax.broadcasted_iota(jnp.int32, (1, 4), 1)

    def write_row(i, k, px, py, pz):
        v4 = jnp.where(l4 == 0, px,
                       jnp.where(l4 == 1, py,
                                 jnp.where(l4 == 2, pz, 0.0)))
        poss_ref[pl.ds(i, 1), :] = v4.astype(jnp.float32)
        idx_ref[pl.ds(i, 1), :] = jnp.full((1, 1), k, jnp.int32)

    k0 = jnp.int32(0)
    px0 = _extract_lane(posx_ref, k0)
    py0 = _extract_lane(posy_ref, k0)
    pz0 = _extract_lane(posz_ref, k0)
    write_row(0, k0, px0, py0, pz0)

    def body(i, carry):
        px, py, pz = carry
        dx = posx_ref[...] - px
        dy = posy_ref[...] - py
        dz = posz_ref[...] - pz
        d = (dx * dx + dy * dy) + dz * dz
        nd = jnp.minimum(dist_ref[...], d)
        dist_ref[...] = nd
        m = jnp.max(nd)
        key = jnp.where(nd == m, ii, _BIG_I32)
        k = jnp.min(key)
        npx = _extract_lane(posx_ref, k)
        npy = _extract_lane(posy_ref, k)
        npz = _extract_lane(posz_ref, k)
        write_row(i, k, npx, npy, npz)
        return (npx, npy, npz)

    lax.fori_loop(1, M_SMP, body, (px0, py0, pz0))


def _fps_call(posx, posy, posz, interpret=False):
    return pl.pallas_call(
        _fps_body,
        out_shape=[
            jax.ShapeDtypeStruct((M_PAD, 4), jnp.float32),
            jax.ShapeDtypeStruct((M_PAD, 1), jnp.int32),
        ],
        scratch_shapes=[pltpu.VMEM((80, 128), jnp.float32)],
        interpret=interpret,
    )(posx, posy, posz)


def _prep_planes(pos):
    pads = ((0, N_PAD - N_PTS),)
    px = jnp.pad(pos[:, 0], pads, constant_values=1e9).reshape(80, 128)
    py = jnp.pad(pos[:, 1], pads, constant_values=1e9).reshape(80, 128)
    pz = jnp.pad(pos[:, 2], pads, constant_values=1e9).reshape(80, 128)
    return px, py, pz


def kernel(x, pos, batch, W1, b1, W2, b2):
    px, py, pz = _prep_planes(pos)
    poss4, idxc = _fps_call(px, py, pz)
    idx = idxc[:M_SMP, 0]
    pos_s = poss4[:M_SMP, :3]
    batch_s = jnp.take(batch, idx, axis=0)

    q2 = jnp.sum(pos_s ** 2, axis=1)[:, None]
    p2 = jnp.sum(pos ** 2, axis=1)[None, :]
    d2 = q2 + p2 - 2.0 * (pos_s @ pos.T)
    within = d2 <= RADIUS * RADIUS
    neg_key = jnp.where(within, -jnp.arange(N_PTS, dtype=jnp.int32),
                        jnp.int32(-N_PTS))
    vals, cols = jax.lax.top_k(neg_key, K_NBR)
    valid = vals > -N_PTS
    x_j = jnp.take(x, cols, axis=0)
    rel = jnp.take(pos, cols, axis=0) - pos_s[:, None, :]
    msg = jnp.concatenate([x_j, rel], axis=-1)
    h = jax.nn.relu(msg @ W1 + b1)
    h = jax.nn.relu(h @ W2 + b2)
    neg = jnp.finfo(h.dtype).min
    h = jnp.where(valid[:, :, None], h, neg)
    out = jnp.max(h, axis=1)
    out = jnp.where(jnp.any(valid, axis=1)[:, None], out, 0.0)
    return out, pos_s, batch_s


# speedup vs baseline: 1.9006x; 1.9006x over previous
import functools

import jax
import jax.numpy as jnp
from jax import lax
from jax.experimental import pallas as pl
from jax.experimental.pallas import tpu as pltpu

N_PTS = 10000
N_PAD = 10240
M_SMP = 2500
M_PAD = 2560
K_NBR = 64
RADIUS = 0.2
D_FEAT = 128
D_HID = 128
_BIG_I32 = 2 ** 30


def _extract_lane(ref, k):
    row = k // 128
    lane = k - row * 128
    rv = ref[pl.ds(row, 1), :]
    li = lax.broadcasted_iota(jnp.int32, (1, 128), 1)
    return jnp.sum(jnp.where(li == lane, rv, 0.0))


def _fps_body(posx_ref, posy_ref, posz_ref, poss_ref, idx_ref, dist_ref):
    ii = (lax.broadcasted_iota(jnp.int32, (80, 128), 0) * 128
          + lax.broadcasted_iota(jnp.int32, (80, 128), 1))
    dist_ref[...] = jnp.where(ii < N_PTS, jnp.inf, -jnp.inf).astype(jnp.float32)

    l4 = lax.broadcasted_iota(jnp.int32, (1, 4), 1)

    def write_row(i, k, px, py, pz):
        v4 = jnp.where(l4 == 0, px,
                       jnp.where(l4 == 1, py,
                                 jnp.where(l4 == 2, pz, 0.0)))
        poss_ref[pl.ds(i, 1), :] = v4.astype(jnp.float32)
        idx_ref[pl.ds(i, 1), :] = jnp.full((1, 1), k, jnp.int32)

    k0 = jnp.int32(0)
    px0 = _extract_lane(posx_ref, k0)
    py0 = _extract_lane(posy_ref, k0)
    pz0 = _extract_lane(posz_ref, k0)
    write_row(0, k0, px0, py0, pz0)

    def body(i, carry):
        px, py, pz = carry
        dx = posx_ref[...] - px
        dy = posy_ref[...] - py
        dz = posz_ref[...] - pz
        d = (dx * dx + dy * dy) + dz * dz
        nd = jnp.minimum(dist_ref[...], d)
        dist_ref[...] = nd
        m = jnp.max(nd)
        key = jnp.where(nd == m, ii, _BIG_I32)
        k = jnp.min(key)
        npx = _extract_lane(posx_ref, k)
        npy = _extract_lane(posy_ref, k)
        npz = _extract_lane(posz_ref, k)
        write_row(i, k, npx, npy, npz)
        return (npx, npy, npz)

    lax.fori_loop(1, M_SMP, body, (px0, py0, pz0))


def _fps_call(posx, posy, posz, interpret=False):
    return pl.pallas_call(
        _fps_body,
        out_shape=[
            jax.ShapeDtypeStruct((M_PAD, 4), jnp.float32),
            jax.ShapeDtypeStruct((M_PAD, 1), jnp.int32),
        ],
        scratch_shapes=[pltpu.VMEM((80, 128), jnp.float32)],
        interpret=interpret,
    )(posx, posy, posz)


def _prep_planes(pos):
    pads = ((0, N_PAD - N_PTS),)
    px = jnp.pad(pos[:, 0], pads, constant_values=1e9).reshape(80, 128)
    py = jnp.pad(pos[:, 1], pads, constant_values=1e9).reshape(80, 128)
    pz = jnp.pad(pos[:, 2], pads, constant_values=1e9).reshape(80, 128)
    return px, py, pz


def kernel(x, pos, batch, W1, b1, W2, b2):
    px, py, pz = _prep_planes(pos)
    poss4, idxc = _fps_call(px, py, pz)
    idx = idxc[:M_SMP, 0]
    pos_s = poss4[:M_SMP, :3]
    batch_s = jnp.take(batch, idx, axis=0)

    q2 = jnp.sum(pos_s ** 2, axis=1)[:, None]
    p2 = jnp.sum(pos ** 2, axis=1)[None, :]
    d2 = q2 + p2 - 2.0 * (pos_s @ pos.T)
    within = d2 <= RADIUS * RADIUS
    neg_key = jnp.where(within, -jnp.arange(N_PTS, dtype=jnp.int32),
                        jnp.int32(-N_PTS))
    vals, cols = jax.lax.top_k(neg_key, K_NBR)
    valid = vals > -N_PTS
    x_j = jnp.take(x, cols, axis=0)
    rel = jnp.take(pos, cols, axis=0) - pos_s[:, None, :]
    msg = jnp.concatenate([x_j, rel], axis=-1)
    h = jax.nn.relu(msg @ W1 + b1)
    h = jax.nn.relu(h @ W2 + b2)
    neg = jnp.finfo(h.dtype).min
    h = jnp.where(valid[:, :, None], h, neg)
    out = jnp.max(h, axis=1)
    out = jnp.where(jnp.any(valid, axis=1)[:, None], out, 0.0)
    return out, pos_s, batch_s


# speedup vs baseline: 25.5721x; 13.4544x over previous
import functools

import jax
import jax.numpy as jnp
from jax import lax
from jax.experimental import pallas as pl
from jax.experimental.pallas import tpu as pltpu
from jax.experimental.pallas import tpu_sc as plsc

N_PTS = 10000
N_PAD = 10240
M_SMP = 2500
M_PAD = 2560
K_NBR = 64
RADIUS = 0.2
R2 = RADIUS * RADIUS
_BIG_I32 = 2 ** 30
_NEG = float(jnp.finfo(jnp.float32).min)

NW = 32
ROWS_PER_W = M_PAD // NW
N_CHUNK = N_PAD // 16


def _extract_lane(ref, k):
    row = k // 128
    lane = k - row * 128
    rv = ref[pl.ds(row, 1), :]
    li = lax.broadcasted_iota(jnp.int32, (1, 128), 1)
    return jnp.sum(jnp.where(li == lane, rv, 0.0))


def _fps_body(posx_ref, posy_ref, posz_ref, poss_ref, idx_ref, dist_ref):
    ii = (lax.broadcasted_iota(jnp.int32, (80, 128), 0) * 128
          + lax.broadcasted_iota(jnp.int32, (80, 128), 1))
    dist_ref[...] = jnp.where(ii < N_PTS, jnp.inf, -jnp.inf).astype(jnp.float32)
    poss_ref[...] = jnp.full((M_PAD, 4), -1e9, jnp.float32)
    idx_ref[...] = jnp.zeros((M_PAD, 1), jnp.int32)

    l4 = lax.broadcasted_iota(jnp.int32, (1, 4), 1)

    def write_row(i, k, px, py, pz):
        v4 = jnp.where(l4 == 0, px,
                       jnp.where(l4 == 1, py,
                                 jnp.where(l4 == 2, pz, 0.0)))
        poss_ref[pl.ds(i, 1), :] = v4.astype(jnp.float32)
        idx_ref[pl.ds(i, 1), :] = jnp.full((1, 1), k, jnp.int32)

    k0 = jnp.int32(0)
    px0 = _extract_lane(posx_ref, k0)
    py0 = _extract_lane(posy_ref, k0)
    pz0 = _extract_lane(posz_ref, k0)
    write_row(0, k0, px0, py0, pz0)

    def body(i, carry):
        px, py, pz = carry
        dx = posx_ref[...] - px
        dy = posy_ref[...] - py
        dz = posz_ref[...] - pz
        d = (dx * dx + dy * dy) + dz * dz
        nd = jnp.minimum(dist_ref[...], d)
        dist_ref[...] = nd
        m = jnp.max(nd)
        key = jnp.where(nd == m, ii, _BIG_I32)
        k = jnp.min(key)
        npx = _extract_lane(posx_ref, k)
        npy = _extract_lane(posy_ref, k)
        npz = _extract_lane(posz_ref, k)
        write_row(i, k, npx, npy, npz)
        return (npx, npy, npz)

    lax.fori_loop(1, M_SMP, body, (px0, py0, pz0))


def _fps_call(posx, posy, posz):
    return pl.pallas_call(
        _fps_body,
        out_shape=[
            jax.ShapeDtypeStruct((M_PAD, 4), jnp.float32),
            jax.ShapeDtypeStruct((M_PAD, 1), jnp.int32),
        ],
        scratch_shapes=[pltpu.VMEM((80, 128), jnp.float32)],
    )(posx, posy, posz)


def _pb_body(x_ref, pos4_ref, w1x_ref, w1p_ref, b1_ref, poss_ref,
             p_ref, b_ref):
    p = jnp.dot(x_ref[...], w1x_ref[...], preferred_element_type=jnp.float32)
    p = p + jnp.dot(pos4_ref[...], w1p_ref[...],
                    preferred_element_type=jnp.float32)
    p_ref[...] = p + b1_ref[...]
    b_ref[...] = jnp.dot(poss_ref[...], w1p_ref[...],
                         preferred_element_type=jnp.float32)


def _pb_call(xpad, pos4, w1x, w1p, b1r, poss4):
    return pl.pallas_call(
        _pb_body,
        out_shape=[
            jax.ShapeDtypeStruct((N_PAD, 128), jnp.float32),
            jax.ShapeDtypeStruct((M_PAD, 128), jnp.float32),
        ],
    )(xpad, pos4, w1x, w1p, b1r, poss4)


def _rne_bf16(v):
    u = plsc.bitcast(v, jnp.int32)
    r = (u + 0x7FFF + ((u >> 16) & 1)) & jnp.int32(-65536)
    return plsc.bitcast(r, jnp.float32)


def _sc_body(posx_hbm, posy_hbm, posz_hbm, qx_hbm, qy_hbm, qz_hbm, p_hbm,
             pg_hbm, cnt_hbm,
             posx_v, posy_v, posz_v, qx_v, qy_v, qz_v,
             cols_v, cnt_v, pg_v, pxb_v, pyb_v, pzb_v, p2_v, sem):
    wid = lax.axis_index("s") * 2 + lax.axis_index("c")
    base = wid * ROWS_PER_W

    pltpu.sync_copy(posx_hbm, posx_v)
    pltpu.sync_copy(posy_hbm, posy_v)
    pltpu.sync_copy(posz_hbm, posz_v)

    def prep(c, _):
        off = c * 16
        vx = posx_v[pl.ds(off, 16)]
        vy = posy_v[pl.ds(off, 16)]
        vz = posz_v[pl.ds(off, 16)]
        pxb_v[pl.ds(off, 16)] = _rne_bf16(vx)
        pyb_v[pl.ds(off, 16)] = _rne_bf16(vy)
        pzb_v[pl.ds(off, 16)] = _rne_bf16(vz)
        p2_v[pl.ds(off, 16)] = (vx * vx + vy * vy) + vz * vz
        return 0

    lax.fori_loop(0, N_CHUNK, prep, 0)
    pltpu.sync_copy(qx_hbm.at[pl.ds(base, ROWS_PER_W)],
                    qx_v.at[pl.ds(0, ROWS_PER_W)])
    pltpu.sync_copy(qy_hbm.at[pl.ds(base, ROWS_PER_W)],
                    qy_v.at[pl.ds(0, ROWS_PER_W)])
    pltpu.sync_copy(qz_hbm.at[pl.ds(base, ROWS_PER_W)],
                    qz_v.at[pl.ds(0, ROWS_PER_W)])

    zeros16 = jnp.zeros((16,), jnp.int32)
    for j in range(128 // 16):
        cnt_v[pl.ds(j * 16, 16)] = zeros16

    iota16 = lax.broadcasted_iota(jnp.int32, (16,), 0)
    lane0 = iota16 == 0

    def row_body(r, _):
        grow = base + r

        @pl.when(grow < M_SMP)
        def _():
            rsplat = jnp.full((16,), r, jnp.int32)
            qx = plsc.load_gather(qx_v, [rsplat])
            qy = plsc.load_gather(qy_v, [rsplat])
            qz = plsc.load_gather(qz_v, [rsplat])
            qxb = _rne_bf16(qx)
            qyb = _rne_bf16(qy)
            qzb = _rne_bf16(qz)
            q2 = (qx * qx + qy * qy) + qz * qz
            for j in range(5):
                cols_v[pl.ds(j * 16, 16)] = zeros16

            def cond(st):
                chunk, count = st
                return (count < K_NBR) & (chunk < N_CHUNK)

            def scan(st):
                chunk, count = st
                off = chunk * 16
                dot = (pxb_v[pl.ds(off, 16)] * qxb
                       + pyb_v[pl.ds(off, 16)] * qyb) \
                    + pzb_v[pl.ds(off, 16)] * qzb
                d2 = (q2 + p2_v[pl.ds(off, 16)]) - 2.0 * dot
                msk = d2 <= R2
                plsc.store_compressed(cols_v.at[pl.ds(count, 16)],
                                      off + iota16, mask=msk)
                c = jnp.sum(msk.astype(jnp.int32))
                return (chunk + 1, count + c)

            _, count = lax.while_loop(cond, scan, (jnp.int32(0), jnp.int32(0)))
            count = jnp.minimum(count, K_NBR)
            plsc.store_scatter(cnt_v, [rsplat],
                               jnp.full((16,), count, jnp.int32), mask=lane0)
            pltpu.async_copy(p_hbm.at[cols_v.at[pl.ds(0, K_NBR)]], pg_v,
                             sem).wait()
            pltpu.sync_copy(pg_v, pg_hbm.at[grow])

        return 0

    lax.fori_loop(0, ROWS_PER_W, row_body, 0)
    pltpu.sync_copy(cnt_v.at[pl.ds(0, ROWS_PER_W)],
                    cnt_hbm.at[pl.ds(base, ROWS_PER_W)])


def _sc_call(posx, posy, posz, qx, qy, qz, p):
    mesh = plsc.VectorSubcoreMesh(core_axis_name="c", subcore_axis_name="s",
                                  num_cores=2, num_subcores=16)
    f = pl.kernel(
        _sc_body,
        out_type=[
            jax.ShapeDtypeStruct((M_PAD, K_NBR, 128), jnp.float32),
            jax.ShapeDtypeStruct((M_PAD,), jnp.int32),
        ],
        mesh=mesh,
        compiler_params=pltpu.CompilerParams(needs_layout_passes=False),
        scratch_types=[
            pltpu.VMEM((N_PAD,), jnp.float32),
            pltpu.VMEM((N_PAD,), jnp.float32),
            pltpu.VMEM((N_PAD,), jnp.float32),
            pltpu.VMEM((128,), jnp.float32),
            pltpu.VMEM((128,), jnp.float32),
            pltpu.VMEM((128,), jnp.float32),
            pltpu.VMEM((K_NBR + 16,), jnp.int32),
            pltpu.VMEM((128,), jnp.int32),
            pltpu.VMEM((K_NBR, 128), jnp.float32),
            pltpu.VMEM((N_PAD,), jnp.float32),
            pltpu.VMEM((N_PAD,), jnp.float32),
            pltpu.VMEM((N_PAD,), jnp.float32),
            pltpu.VMEM((N_PAD,), jnp.float32),
            pltpu.SemaphoreType.DMA,
        ],
    )
    return f(posx, posy, posz, qx, qy, qz, p)


QB = 128
GRID = M_PAD // QB


def _mlp_body(pg_ref, bq_ref, cnt_ref, w2_ref, b2_ref, out_ref):
    pg = pg_ref[...]
    bq = bq_ref[...]
    h1 = jnp.maximum(pg - bq[:, None, :], 0.0)
    h1f = h1.reshape(QB * K_NBR, 128)
    h2 = jnp.dot(h1f, w2_ref[...], preferred_element_type=jnp.float32)
    h2 = jnp.maximum(h2 + b2_ref[...], 0.0)
    h2 = h2.reshape(QB, K_NBR, 128)
    cnt = cnt_ref[...]
    slot = lax.broadcasted_iota(jnp.int32, (QB, K_NBR, 128), 1)
    h2m = jnp.where(slot < cnt[:, :, None], h2, _NEG)
    mx = jnp.max(h2m, axis=1)
    out_ref[...] = jnp.where(cnt > 0, mx, 0.0)


def _mlp_call(pg, bq, cnt2, w2, b2r):
    return pl.pallas_call(
        _mlp_body,
        grid=(GRID,),
        in_specs=[
            pl.BlockSpec((QB, K_NBR, 128), lambda g: (g, 0, 0)),
            pl.BlockSpec((QB, 128), lambda g: (g, 0)),
            pl.BlockSpec((QB, 1), lambda g: (g, 0)),
            pl.BlockSpec((128, 128), lambda g: (0, 0)),
            pl.BlockSpec((1, 128), lambda g: (0, 0)),
        ],
        out_specs=pl.BlockSpec((QB, 128), lambda g: (g, 0)),
        out_shape=jax.ShapeDtypeStruct((M_PAD, 128), jnp.float32),
    )(pg, bq, cnt2, w2, b2r)


def _prep_planes(pos):
    pads = ((0, N_PAD - N_PTS),)
    px = jnp.pad(pos[:, 0], pads, constant_values=1e9).reshape(80, 128)
    py = jnp.pad(pos[:, 1], pads, constant_values=1e9).reshape(80, 128)
    pz = jnp.pad(pos[:, 2], pads, constant_values=1e9).reshape(80, 128)
    return px, py, pz


def kernel(x, pos, batch, W1, b1, W2, b2):
    px, py, pz = _prep_planes(pos)
    poss4, idxc = _fps_call(px, py, pz)
    idx = idxc[:M_SMP, 0]
    pos_s = poss4[:M_SMP, :3]
    batch_s = jnp.take(batch, idx, axis=0)

    xpad = jnp.pad(x, ((0, N_PAD - N_PTS), (0, 0)))
    pos4 = jnp.pad(pos, ((0, N_PAD - N_PTS), (0, 1)))
    w1x = W1[:128]
    w1p = jnp.pad(W1[128:], ((0, 1), (0, 0)))
    b1r = b1.reshape(1, 128)
    p_all, bq = _pb_call(xpad, pos4, w1x, w1p, b1r, poss4)

    pg, cnt = _sc_call(px.reshape(N_PAD), py.reshape(N_PAD),
                       pz.reshape(N_PAD), poss4[:, 0], poss4[:, 1],
                       poss4[:, 2], p_all)

    out_pad = _mlp_call(pg, bq, cnt.reshape(M_PAD, 1), W2,
                        b2.reshape(1, 128))
    return out_pad[:M_SMP], pos_s, batch_s


# speedup vs baseline: 32.6589x; 1.2771x over previous
import functools

import jax
import jax.numpy as jnp
from jax import lax
from jax.experimental import pallas as pl
from jax.experimental.pallas import tpu as pltpu
from jax.experimental.pallas import tpu_sc as plsc

N_PTS = 10000
N_PAD = 10240
M_SMP = 2500
M_PAD = 2560
K_NBR = 64
RADIUS = 0.2
R2 = RADIUS * RADIUS
_BIG_I32 = 2 ** 30
_NEG = float(jnp.finfo(jnp.float32).min)

NW = 32
ROWS_PER_W = M_PAD // NW
N_CHUNK = N_PAD // 16


def _fps_body(posx_ref, posy_ref, posz_ref, posxc_ref, posyc_ref, poszc_ref,
              poss_ref, idx_ref, dist_ref):
    ii = (lax.broadcasted_iota(jnp.int32, (80, 128), 0) * 128
          + lax.broadcasted_iota(jnp.int32, (80, 128), 1))
    iif = ii.astype(jnp.float32)
    dist_ref[...] = jnp.where(ii < N_PTS, jnp.inf, -jnp.inf).astype(jnp.float32)
    poss_ref[...] = jnp.full((M_PAD, 4), -1e9, jnp.float32)
    idx_ref[...] = jnp.zeros((M_PAD, 1), jnp.int32)

    l4 = lax.broadcasted_iota(jnp.int32, (1, 4), 1)

    def write_row(i, k, px, py, pz):
        v4 = jnp.where(l4 == 0, px,
                       jnp.where(l4 == 1, py,
                                 jnp.where(l4 == 2, pz, 0.0)))
        poss_ref[pl.ds(i, 1), :] = v4.astype(jnp.float32)
        idx_ref[pl.ds(i, 1), :] = jnp.full((1, 1), k, jnp.int32)

    def extract(k):
        return (posxc_ref[k, 0], posyc_ref[k, 0], poszc_ref[k, 0])

    px0, py0, pz0 = extract(jnp.int32(0))
    write_row(0, jnp.int32(0), px0, py0, pz0)

    def body(i, carry):
        px, py, pz = carry
        dx = posx_ref[...] - px
        dy = posy_ref[...] - py
        dz = posz_ref[...] - pz
        d = (dx * dx + dy * dy) + dz * dz
        nd = jnp.minimum(dist_ref[...], d)
        dist_ref[...] = nd
        m = jnp.max(nd)
        key = jnp.where(nd == m, iif, 3.0e38)
        k = jnp.min(key).astype(jnp.int32)
        npx, npy, npz = extract(k)
        write_row(i, k, npx, npy, npz)
        return (npx, npy, npz)

    lax.fori_loop(1, M_SMP, body, (px0, py0, pz0))


def _fps_call(posx, posy, posz, interpret=False):
    return pl.pallas_call(
        _fps_body,
        out_shape=[
            jax.ShapeDtypeStruct((M_PAD, 4), jnp.float32),
            jax.ShapeDtypeStruct((M_PAD, 1), jnp.int32),
        ],
        scratch_shapes=[pltpu.VMEM((80, 128), jnp.float32)],
        interpret=interpret,
    )(posx, posy, posz,
      posx.reshape(N_PAD, 1), posy.reshape(N_PAD, 1), posz.reshape(N_PAD, 1))


def _pb_body(x_ref, pos4_ref, w1x_ref, w1p_ref, b1_ref, poss_ref,
             p_ref, b_ref):
    p = jnp.dot(x_ref[...], w1x_ref[...], preferred_element_type=jnp.float32)
    p = p + jnp.dot(pos4_ref[...], w1p_ref[...],
                    preferred_element_type=jnp.float32)
    p_ref[...] = p + b1_ref[...]
    b_ref[...] = jnp.dot(poss_ref[...], w1p_ref[...],
                         preferred_element_type=jnp.float32)


def _pb_call(xpad, pos4, w1x, w1p, b1r, poss4):
    return pl.pallas_call(
        _pb_body,
        out_shape=[
            jax.ShapeDtypeStruct((N_PAD, 128), jnp.float32),
            jax.ShapeDtypeStruct((M_PAD, 128), jnp.float32),
        ],
    )(xpad, pos4, w1x, w1p, b1r, poss4)


def _rne_bf16(v):
    u = plsc.bitcast(v, jnp.int32)
    r = (u + 0x7FFF + ((u >> 16) & 1)) & jnp.int32(-65536)
    return plsc.bitcast(r, jnp.float32)


def _sc_body(posx_hbm, posy_hbm, posz_hbm, qx_hbm, qy_hbm, qz_hbm, p_hbm,
             pg_hbm, cnt_hbm,
             posx_v, posy_v, posz_v, qx_v, qy_v, qz_v,
             cols_v, cnt_v, pg_v, pxb_v, pyb_v, pzb_v, p2_v, sem):
    wid = lax.axis_index("s") * 2 + lax.axis_index("c")
    base = wid * ROWS_PER_W

    pltpu.sync_copy(posx_hbm, posx_v)
    pltpu.sync_copy(posy_hbm, posy_v)
    pltpu.sync_copy(posz_hbm, posz_v)

    def prep(c, _):
        off = c * 16
        vx = posx_v[pl.ds(off, 16)]
        vy = posy_v[pl.ds(off, 16)]
        vz = posz_v[pl.ds(off, 16)]
        pxb_v[pl.ds(off, 16)] = _rne_bf16(vx)
        pyb_v[pl.ds(off, 16)] = _rne_bf16(vy)
        pzb_v[pl.ds(off, 16)] = _rne_bf16(vz)
        p2_v[pl.ds(off, 16)] = (vx * vx + vy * vy) + vz * vz
        return 0

    lax.fori_loop(0, N_CHUNK, prep, 0)
    pltpu.sync_copy(qx_hbm.at[pl.ds(base, ROWS_PER_W)],
                    qx_v.at[pl.ds(0, ROWS_PER_W)])
    pltpu.sync_copy(qy_hbm.at[pl.ds(base, ROWS_PER_W)],
                    qy_v.at[pl.ds(0, ROWS_PER_W)])
    pltpu.sync_copy(qz_hbm.at[pl.ds(base, ROWS_PER_W)],
                    qz_v.at[pl.ds(0, ROWS_PER_W)])

    zeros16 = jnp.zeros((16,), jnp.int32)
    for j in range(128 // 16):
        cnt_v[pl.ds(j * 16, 16)] = zeros16

    iota16 = lax.broadcasted_iota(jnp.int32, (16,), 0)
    lane0 = iota16 == 0

    def row_body(r, _):
        grow = base + r

        @pl.when(grow < M_SMP)
        def _():
            rsplat = jnp.full((16,), r, jnp.int32)
            qx = plsc.load_gather(qx_v, [rsplat])
            qy = plsc.load_gather(qy_v, [rsplat])
            qz = plsc.load_gather(qz_v, [rsplat])
            qxb = _rne_bf16(qx)
            qyb = _rne_bf16(qy)
            qzb = _rne_bf16(qz)
            q2 = (qx * qx + qy * qy) + qz * qz
            for j in range(5):
                cols_v[pl.ds(j * 16, 16)] = zeros16

            def cond(st):
                chunk, count = st
                return (count < K_NBR) & (chunk < N_CHUNK)

            def scan(st):
                chunk, count = st
                off = chunk * 16
                dot = (pxb_v[pl.ds(off, 16)] * qxb
                       + pyb_v[pl.ds(off, 16)] * qyb) \
                    + pzb_v[pl.ds(off, 16)] * qzb
                d2 = (q2 + p2_v[pl.ds(off, 16)]) - 2.0 * dot
                msk = d2 <= R2
                plsc.store_compressed(cols_v.at[pl.ds(count, 16)],
                                      off + iota16, mask=msk)
                c = jnp.sum(msk.astype(jnp.int32))
                return (chunk + 1, count + c)

            _, count = lax.while_loop(cond, scan, (jnp.int32(0), jnp.int32(0)))
            count = jnp.minimum(count, K_NBR)
            plsc.store_scatter(cnt_v, [rsplat],
                               jnp.full((16,), count, jnp.int32), mask=lane0)
            pltpu.async_copy(p_hbm.at[cols_v.at[pl.ds(0, K_NBR)]], pg_v,
                             sem).wait()
            pltpu.sync_copy(pg_v, pg_hbm.at[grow])

        return 0

    lax.fori_loop(0, ROWS_PER_W, row_body, 0)
    pltpu.sync_copy(cnt_v.at[pl.ds(0, ROWS_PER_W)],
                    cnt_hbm.at[pl.ds(base, ROWS_PER_W)])


def _sc_call(posx, posy, posz, qx, qy, qz, p):
    mesh = plsc.VectorSubcoreMesh(core_axis_name="c", subcore_axis_name="s",
                                  num_cores=2, num_subcores=16)
    f = pl.kernel(
        _sc_body,
        out_type=[
            jax.ShapeDtypeStruct((M_PAD, K_NBR, 128), jnp.float32),
            jax.ShapeDtypeStruct((M_PAD,), jnp.int32),
        ],
        mesh=mesh,
        compiler_params=pltpu.CompilerParams(needs_layout_passes=False),
        scratch_types=[
            pltpu.VMEM((N_PAD,), jnp.float32),
            pltpu.VMEM((N_PAD,), jnp.float32),
            pltpu.VMEM((N_PAD,), jnp.float32),
            pltpu.VMEM((128,), jnp.float32),
            pltpu.VMEM((128,), jnp.float32),
            pltpu.VMEM((128,), jnp.float32),
            pltpu.VMEM((K_NBR + 16,), jnp.int32),
            pltpu.VMEM((128,), jnp.int32),
            pltpu.VMEM((K_NBR, 128), jnp.float32),
            pltpu.VMEM((N_PAD,), jnp.float32),
            pltpu.VMEM((N_PAD,), jnp.float32),
            pltpu.VMEM((N_PAD,), jnp.float32),
            pltpu.VMEM((N_PAD,), jnp.float32),
            pltpu.SemaphoreType.DMA,
        ],
    )
    return f(posx, posy, posz, qx, qy, qz, p)


QB = 128
GRID = M_PAD // QB


def _mlp_body(pg_ref, bq_ref, cnt_ref, w2_ref, b2_ref, out_ref):
    pg = pg_ref[...]
    bq = bq_ref[...]
    h1 = jnp.maximum(pg - bq[:, None, :], 0.0)
    h1f = h1.reshape(QB * K_NBR, 128)
    h2 = jnp.dot(h1f, w2_ref[...], preferred_element_type=jnp.float32)
    h2 = jnp.maximum(h2 + b2_ref[...], 0.0)
    h2 = h2.reshape(QB, K_NBR, 128)
    cnt = cnt_ref[...]
    slot = lax.broadcasted_iota(jnp.int32, (QB, K_NBR, 128), 1)
    h2m = jnp.where(slot < cnt[:, :, None], h2, _NEG)
    mx = jnp.max(h2m, axis=1)
    out_ref[...] = jnp.where(cnt > 0, mx, 0.0)


def _mlp_call(pg, bq, cnt2, w2, b2r):
    return pl.pallas_call(
        _mlp_body,
        grid=(GRID,),
        in_specs=[
            pl.BlockSpec((QB, K_NBR, 128), lambda g: (g, 0, 0)),
            pl.BlockSpec((QB, 128), lambda g: (g, 0)),
            pl.BlockSpec((QB, 1), lambda g: (g, 0)),
            pl.BlockSpec((128, 128), lambda g: (0, 0)),
            pl.BlockSpec((1, 128), lambda g: (0, 0)),
        ],
        out_specs=pl.BlockSpec((QB, 128), lambda g: (g, 0)),
        out_shape=jax.ShapeDtypeStruct((M_PAD, 128), jnp.float32),
    )(pg, bq, cnt2, w2, b2r)


def _prep_planes(pos):
    pads = ((0, N_PAD - N_PTS),)
    px = jnp.pad(pos[:, 0], pads, constant_values=1e9).reshape(80, 128)
    py = jnp.pad(pos[:, 1], pads, constant_values=1e9).reshape(80, 128)
    pz = jnp.pad(pos[:, 2], pads, constant_values=1e9).reshape(80, 128)
    return px, py, pz


def kernel(x, pos, batch, W1, b1, W2, b2):
    px, py, pz = _prep_planes(pos)
    poss4, idxc = _fps_call(px, py, pz)
    idx = idxc[:M_SMP, 0]
    pos_s = poss4[:M_SMP, :3]
    batch_s = jnp.take(batch, idx, axis=0)

    xpad = jnp.pad(x, ((0, N_PAD - N_PTS), (0, 0)))
    pos4 = jnp.pad(pos, ((0, N_PAD - N_PTS), (0, 1)))
    w1x = W1[:128]
    w1p = jnp.pad(W1[128:], ((0, 1), (0, 0)))
    b1r = b1.reshape(1, 128)
    p_all, bq = _pb_call(xpad, pos4, w1x, w1p, b1r, poss4)

    pg, cnt = _sc_call(px.reshape(N_PAD), py.reshape(N_PAD),
                       pz.reshape(N_PAD), poss4[:, 0], poss4[:, 1],
                       poss4[:, 2], p_all)

    out_pad = _mlp_call(pg, bq, cnt.reshape(M_PAD, 1), W2,
                        b2.reshape(1, 128))
    return out_pad[:M_SMP], pos_s, batch_s


# speedup vs baseline: 37.6082x; 1.1515x over previous
import functools

import jax
import jax.numpy as jnp
from jax import lax
from jax.experimental import pallas as pl
from jax.experimental.pallas import tpu as pltpu
from jax.experimental.pallas import tpu_sc as plsc

N_PTS = 10000
N_PAD = 10240
M_SMP = 2500
M_PAD = 2560
K_NBR = 64
RADIUS = 0.2
R2 = RADIUS * RADIUS
_BIG_I32 = 2 ** 30
_NEG = float(jnp.finfo(jnp.float32).min)

NW = 32
ROWS_PER_W = M_PAD // NW
N_CHUNK = N_PAD // 16


def _fps_body(posx_ref, posy_ref, posz_ref, posxc_ref, posyc_ref, poszc_ref,
              poss_ref, idx_ref, dist_ref):
    ii = (lax.broadcasted_iota(jnp.int32, (80, 128), 0) * 128
          + lax.broadcasted_iota(jnp.int32, (80, 128), 1))
    iif = ii.astype(jnp.float32)
    dist_ref[...] = jnp.where(ii < N_PTS, jnp.inf, -jnp.inf).astype(jnp.float32)
    poss_ref[...] = jnp.full((M_PAD, 4), -1e9, jnp.float32)
    idx_ref[...] = jnp.zeros((M_PAD, 1), jnp.int32)

    l4 = lax.broadcasted_iota(jnp.int32, (1, 4), 1)

    def write_row(i, k, px, py, pz):
        v4 = jnp.where(l4 == 0, px,
                       jnp.where(l4 == 1, py,
                                 jnp.where(l4 == 2, pz, 0.0)))
        poss_ref[pl.ds(i, 1), :] = v4.astype(jnp.float32)
        idx_ref[pl.ds(i, 1), :] = jnp.full((1, 1), k, jnp.int32)

    def extract(k):
        return (posxc_ref[k, 0], posyc_ref[k, 0], poszc_ref[k, 0])

    px0, py0, pz0 = extract(jnp.int32(0))
    write_row(0, jnp.int32(0), px0, py0, pz0)

    def body(i, carry):
        px, py, pz = carry
        dx = posx_ref[...] - px
        dy = posy_ref[...] - py
        dz = posz_ref[...] - pz
        d = (dx * dx + dy * dy) + dz * dz
        nd = jnp.minimum(dist_ref[...], d)
        dist_ref[...] = nd
        m = jnp.max(nd)
        key = jnp.where(nd == m, iif, 3.0e38)
        k = jnp.min(key).astype(jnp.int32)
        npx, npy, npz = extract(k)
        write_row(i, k, npx, npy, npz)
        return (npx, npy, npz)

    lax.fori_loop(1, M_SMP, body, (px0, py0, pz0))


def _fps_call(posx, posy, posz, interpret=False):
    return pl.pallas_call(
        _fps_body,
        out_shape=[
            jax.ShapeDtypeStruct((M_PAD, 4), jnp.float32),
            jax.ShapeDtypeStruct((M_PAD, 1), jnp.int32),
        ],
        scratch_shapes=[pltpu.VMEM((80, 128), jnp.float32)],
        interpret=interpret,
    )(posx, posy, posz,
      posx.reshape(N_PAD, 1), posy.reshape(N_PAD, 1), posz.reshape(N_PAD, 1))


def _pb_body(x_ref, pos4_ref, w1x_ref, w1p_ref, b1_ref, poss_ref,
             p_ref, b_ref):
    p = jnp.dot(x_ref[...], w1x_ref[...], preferred_element_type=jnp.float32)
    p = p + jnp.dot(pos4_ref[...], w1p_ref[...],
                    preferred_element_type=jnp.float32)
    p_ref[...] = p + b1_ref[...]
    b_ref[...] = jnp.dot(poss_ref[...], w1p_ref[...],
                         preferred_element_type=jnp.float32)


def _pb_call(xpad, pos4, w1x, w1p, b1r, poss4):
    return pl.pallas_call(
        _pb_body,
        out_shape=[
            jax.ShapeDtypeStruct((N_PAD, 128), jnp.float32),
            jax.ShapeDtypeStruct((M_PAD, 128), jnp.float32),
        ],
    )(xpad, pos4, w1x, w1p, b1r, poss4)


def _rne_bf16(v):
    u = plsc.bitcast(v, jnp.int32)
    r = (u + 0x7FFF + ((u >> 16) & 1)) & jnp.int32(-65536)
    return plsc.bitcast(r, jnp.float32)


def _sc_body(posx_hbm, posy_hbm, posz_hbm, qx_hbm, qy_hbm, qz_hbm, p_hbm,
             pg_hbm, cnt_hbm,
             posx_v, posy_v, posz_v, qx_v, qy_v, qz_v,
             cols_v, cnt_v, pg_v, pxb_v, pyb_v, pzb_v, p2_v, sem):
    wid = lax.axis_index("s") * 2 + lax.axis_index("c")
    base = wid * ROWS_PER_W

    pltpu.sync_copy(posx_hbm, posx_v)
    pltpu.sync_copy(posy_hbm, posy_v)
    pltpu.sync_copy(posz_hbm, posz_v)

    def prep(c, _):
        off = c * 16
        vx = posx_v[pl.ds(off, 16)]
        vy = posy_v[pl.ds(off, 16)]
        vz = posz_v[pl.ds(off, 16)]
        pxb_v[pl.ds(off, 16)] = _rne_bf16(vx)
        pyb_v[pl.ds(off, 16)] = _rne_bf16(vy)
        pzb_v[pl.ds(off, 16)] = _rne_bf16(vz)
        p2_v[pl.ds(off, 16)] = (vx * vx + vy * vy) + vz * vz
        return 0

    lax.fori_loop(0, N_CHUNK, prep, 0)
    pltpu.sync_copy(qx_hbm.at[pl.ds(base, ROWS_PER_W)],
                    qx_v.at[pl.ds(0, ROWS_PER_W)])
    pltpu.sync_copy(qy_hbm.at[pl.ds(base, ROWS_PER_W)],
                    qy_v.at[pl.ds(0, ROWS_PER_W)])
    pltpu.sync_copy(qz_hbm.at[pl.ds(base, ROWS_PER_W)],
                    qz_v.at[pl.ds(0, ROWS_PER_W)])

    zeros16 = jnp.zeros((16,), jnp.int32)
    for j in range(128 // 16):
        cnt_v[pl.ds(j * 16, 16)] = zeros16

    iota16 = lax.broadcasted_iota(jnp.int32, (16,), 0)
    lane0 = iota16 == 0

    def row_body(r, _):
        grow = base + r

        @pl.when(grow < M_SMP)
        def _():
            rsplat = jnp.full((16,), r, jnp.int32)
            qx = plsc.load_gather(qx_v, [rsplat])
            qy = plsc.load_gather(qy_v, [rsplat])
            qz = plsc.load_gather(qz_v, [rsplat])
            qxb = _rne_bf16(qx)
            qyb = _rne_bf16(qy)
            qzb = _rne_bf16(qz)
            q2 = (qx * qx + qy * qy) + qz * qz
            for j in range(5):
                cols_v[pl.ds(j * 16, 16)] = zeros16

            def cond(st):
                chunk, count = st
                return (count < K_NBR) & (chunk < N_CHUNK)

            def scan(st):
                chunk, count = st
                for u in range(4):
                    off = (chunk + u) * 16
                    dot = (pxb_v[pl.ds(off, 16)] * qxb
                           + pyb_v[pl.ds(off, 16)] * qyb) \
                        + pzb_v[pl.ds(off, 16)] * qzb
                    d2 = (q2 + p2_v[pl.ds(off, 16)]) - 2.0 * dot
                    msk = d2 <= R2
                    plsc.store_compressed(cols_v.at[pl.ds(count, 16)],
                                          off + iota16, mask=msk)
                    count = count + jnp.sum(msk.astype(jnp.int32))
                return (chunk + 4, count)

            _, count = lax.while_loop(cond, scan, (jnp.int32(0), jnp.int32(0)))
            count = jnp.minimum(count, K_NBR)
            plsc.store_scatter(cnt_v, [rsplat],
                               jnp.full((16,), count, jnp.int32), mask=lane0)
            pltpu.async_copy(p_hbm.at[cols_v.at[pl.ds(0, K_NBR)]], pg_v,
                             sem).wait()
            pltpu.sync_copy(pg_v, pg_hbm.at[grow])

        return 0

    lax.fori_loop(0, ROWS_PER_W, row_body, 0)
    pltpu.sync_copy(cnt_v.at[pl.ds(0, ROWS_PER_W)],
                    cnt_hbm.at[pl.ds(base, ROWS_PER_W)])


def _sc_call(posx, posy, posz, qx, qy, qz, p):
    mesh = plsc.VectorSubcoreMesh(core_axis_name="c", subcore_axis_name="s",
                                  num_cores=2, num_subcores=16)
    f = pl.kernel(
        _sc_body,
        out_type=[
            jax.ShapeDtypeStruct((M_PAD, K_NBR, 128), jnp.float32),
            jax.ShapeDtypeStruct((M_PAD,), jnp.int32),
        ],
        mesh=mesh,
        compiler_params=pltpu.CompilerParams(needs_layout_passes=False),
        scratch_types=[
            pltpu.VMEM((N_PAD,), jnp.float32),
            pltpu.VMEM((N_PAD,), jnp.float32),
            pltpu.VMEM((N_PAD,), jnp.float32),
            pltpu.VMEM((128,), jnp.float32),
            pltpu.VMEM((128,), jnp.float32),
            pltpu.VMEM((128,), jnp.float32),
            pltpu.VMEM((128,), jnp.int32),
            pltpu.VMEM((128,), jnp.int32),
            pltpu.VMEM((K_NBR, 128), jnp.float32),
            pltpu.VMEM((N_PAD,), jnp.float32),
            pltpu.VMEM((N_PAD,), jnp.float32),
            pltpu.VMEM((N_PAD,), jnp.float32),
            pltpu.VMEM((N_PAD,), jnp.float32),
            pltpu.SemaphoreType.DMA,
        ],
    )
    return f(posx, posy, posz, qx, qy, qz, p)


QB = 128
GRID = M_PAD // QB


def _mlp_body(pg_ref, bq_ref, cnt_ref, w2_ref, b2_ref, out_ref):
    pg = pg_ref[...]
    bq = bq_ref[...]
    h1 = jnp.maximum(pg - bq[:, None, :], 0.0)
    h1f = h1.reshape(QB * K_NBR, 128)
    h2 = jnp.dot(h1f, w2_ref[...], preferred_element_type=jnp.float32)
    h2 = jnp.maximum(h2 + b2_ref[...], 0.0)
    h2 = h2.reshape(QB, K_NBR, 128)
    cnt = cnt_ref[...]
    slot = lax.broadcasted_iota(jnp.int32, (QB, K_NBR, 128), 1)
    h2m = jnp.where(slot < cnt[:, :, None], h2, _NEG)
    mx = jnp.max(h2m, axis=1)
    out_ref[...] = jnp.where(cnt > 0, mx, 0.0)


def _mlp_call(pg, bq, cnt2, w2, b2r):
    return pl.pallas_call(
        _mlp_body,
        grid=(GRID,),
        in_specs=[
            pl.BlockSpec((QB, K_NBR, 128), lambda g: (g, 0, 0)),
            pl.BlockSpec((QB, 128), lambda g: (g, 0)),
            pl.BlockSpec((QB, 1), lambda g: (g, 0)),
            pl.BlockSpec((128, 128), lambda g: (0, 0)),
            pl.BlockSpec((1, 128), lambda g: (0, 0)),
        ],
        out_specs=pl.BlockSpec((QB, 128), lambda g: (g, 0)),
        out_shape=jax.ShapeDtypeStruct((M_PAD, 128), jnp.float32),
    )(pg, bq, cnt2, w2, b2r)


def _prep_planes(pos):
    pads = ((0, N_PAD - N_PTS),)
    px = jnp.pad(pos[:, 0], pads, constant_values=1e9).reshape(80, 128)
    py = jnp.pad(pos[:, 1], pads, constant_values=1e9).reshape(80, 128)
    pz = jnp.pad(pos[:, 2], pads, constant_values=1e9).reshape(80, 128)
    return px, py, pz


def kernel(x, pos, batch, W1, b1, W2, b2):
    px, py, pz = _prep_planes(pos)
    poss4, idxc = _fps_call(px, py, pz)
    idx = idxc[:M_SMP, 0]
    pos_s = poss4[:M_SMP, :3]
    batch_s = jnp.take(batch, idx, axis=0)

    xpad = jnp.pad(x, ((0, N_PAD - N_PTS), (0, 0)))
    pos4 = jnp.pad(pos, ((0, N_PAD - N_PTS), (0, 1)))
    w1x = W1[:128]
    w1p = jnp.pad(W1[128:], ((0, 1), (0, 0)))
    b1r = b1.reshape(1, 128)
    p_all, bq = _pb_call(xpad, pos4, w1x, w1p, b1r, poss4)

    pg, cnt = _sc_call(px.reshape(N_PAD), py.reshape(N_PAD),
                       pz.reshape(N_PAD), poss4[:, 0], poss4[:, 1],
                       poss4[:, 2], p_all)

    out_pad = _mlp_call(pg, bq, cnt.reshape(M_PAD, 1), W2,
                        b2.reshape(1, 128))
    return out_pad[:M_SMP], pos_s, batch_s
